# baseline (speedup 1.0000x reference)
import jax
import jax.numpy as jnp
from jax.experimental import pallas as pl
from jax.experimental.pallas import tpu as pltpu

IN_FEATS = 2048
OUT = 10
CHUNKS = 16
THRESH = 0.05
CHUNK_DIM = IN_FEATS // CHUNKS
N_TOK = 16384

GRP = 16
WIDE = CHUNKS * GRP
BLK = 512
NSTEPS = N_TOK // BLK
BLK2 = 4096


def _main_kernel(x_ref, ew_ref, rw_ref, eb_ref, rb_ref, m10_ref, mr_ref,
                 rout_ref, outpre_ref, acts_ref, wscr, bscr):
    i = pl.program_id(0)

    @pl.when(i == 0)
    def _prologue():
        lane = jax.lax.broadcasted_iota(jnp.int32, (1, WIDE), 1)
        rowc = jax.lax.broadcasted_iota(jnp.int32, (IN_FEATS, 1), 0) // CHUNK_DIM
        spread = jnp.dot(ew_ref[:], m10_ref[:],
                         preferred_element_type=jnp.float32)
        blockmask = (lane % GRP) == rowc
        wscr[:] = (jnp.where(blockmask, spread, 0.0)
                   + jnp.dot(rw_ref[:], mr_ref[:],
                             preferred_element_type=jnp.float32))
        ebs = jnp.dot(eb_ref[:], m10_ref[:],
                      preferred_element_type=jnp.float32)
        crow = jax.lax.broadcasted_iota(jnp.int32, (CHUNKS, 1), 0)
        ebv = jnp.sum(jnp.where((lane % GRP) == crow, ebs, 0.0),
                      axis=0, keepdims=True)
        bscr[:] = ebv + jnp.dot(rb_ref[:], mr_ref[:],
                                preferred_element_type=jnp.float32)
        acts_ref[:] = jnp.zeros_like(acts_ref)

    @pl.when(i > 0)
    def _body():
        acc = jnp.dot(x_ref[:], wscr[:], preferred_element_type=jnp.float32)
        acc = acc + bscr[:]

        logits = acc[:, CHUNKS * (GRP - 1):]
        ci = jax.lax.broadcasted_iota(jnp.int32, (1, CHUNKS), 1)
        m = jnp.max(logits, axis=1, keepdims=True)
        e = jnp.exp(logits - m)
        s = jnp.sum(e, axis=1, keepdims=True)
        p = e / s
        v1 = jnp.max(p, axis=1, keepdims=True)
        l1 = jnp.min(jnp.where(p == v1, ci, CHUNKS), axis=1, keepdims=True)
        p2 = jnp.where(ci == l1, -1.0, p)
        v2 = jnp.max(p2, axis=1, keepdims=True)
        l2 = jnp.min(jnp.where(p2 == v2, ci, CHUNKS), axis=1, keepdims=True)
        g16 = (jnp.where(ci == l1, v1, 0.0)
               + jnp.where(ci == l2, v2, 0.0))

        g = jnp.concatenate([g16] * GRP, axis=1)
        outpre_ref[:] = jnp.dot(g * acc, rout_ref[:],
                                preferred_element_type=jnp.float32)

        acts_ref[:] = acts_ref[:] + jnp.sum(jnp.abs(acc), axis=0,
                                            keepdims=True)


def _head_kernel(outpre_ref, acts_ref, rchunk_ref, qw_ref, qb_ref, o_ref):
    acts16 = jnp.dot(acts_ref[:], rchunk_ref[:],
                     preferred_element_type=jnp.float32)
    scalar_act = jnp.max(acts16) * (1.0 / (N_TOK * OUT))
    qw = qw_ref[:]
    mean_abs = jnp.sum(jnp.abs(qw)) * (1.0 / (OUT * OUT))
    wq = jnp.where(scalar_act > THRESH, qw, jnp.sign(qw) * mean_abs)
    res = jnp.dot(outpre_ref[:], wq, preferred_element_type=jnp.float32)
    res = res + qb_ref[:]
    o_ref[:] = res[:, :OUT]


def kernel(x, router_w, router_b, expert_w, expert_b, quant_w, quant_b):
    f32 = jnp.float32
    lane = jnp.arange(WIDE)
    o_of = lane // GRP
    c_of = lane % GRP

    m10 = (o_of[None, :] == jnp.arange(OUT)[:, None]).astype(f32)
    mr = (lane[None, :] == (CHUNKS * (GRP - 1) + jnp.arange(CHUNKS))[:, None]
          ).astype(f32)
    real = o_of < OUT
    r_out = ((o_of[:, None] == jnp.arange(CHUNKS)[None, :]) & real[:, None]
             ).astype(f32)
    r_chunk = ((c_of[:, None] == jnp.arange(CHUNKS)[None, :]) & real[:, None]
               ).astype(f32)

    ew2 = expert_w.reshape(IN_FEATS, OUT)
    rb2 = router_b.reshape(1, CHUNKS)
    qw_p = jnp.pad(quant_w, ((0, GRP - OUT), (0, GRP - OUT)))
    qb_p = jnp.pad(quant_b, (0, GRP - OUT)).reshape(1, GRP)

    full = lambda shape: pl.BlockSpec(shape, lambda i: tuple(0 for _ in shape))
    out_pre, acts = pl.pallas_call(
        _main_kernel,
        grid=(NSTEPS + 1,),
        in_specs=[
            pl.BlockSpec((BLK, IN_FEATS), lambda i: (jnp.maximum(i - 1, 0), 0)),
            full((IN_FEATS, OUT)),
            full((IN_FEATS, CHUNKS)),
            full((CHUNKS, OUT)),
            full((1, CHUNKS)),
            full((OUT, WIDE)),
            full((CHUNKS, WIDE)),
            full((WIDE, CHUNKS)),
        ],
        out_specs=[
            pl.BlockSpec((BLK, CHUNKS), lambda i: (jnp.maximum(i - 1, 0), 0)),
            pl.BlockSpec((1, WIDE), lambda i: (0, 0)),
        ],
        out_shape=[
            jax.ShapeDtypeStruct((N_TOK, CHUNKS), f32),
            jax.ShapeDtypeStruct((1, WIDE), f32),
        ],
        scratch_shapes=[
            pltpu.VMEM((IN_FEATS, WIDE), f32),
            pltpu.VMEM((1, WIDE), f32),
        ],
    )(x, ew2, router_w, expert_b, rb2, m10, mr, r_out)

    out = pl.pallas_call(
        _head_kernel,
        grid=(N_TOK // BLK2,),
        in_specs=[
            pl.BlockSpec((BLK2, CHUNKS), lambda i: (i, 0)),
            full((1, WIDE)),
            full((WIDE, CHUNKS)),
            full((GRP, GRP)),
            full((1, GRP)),
        ],
        out_specs=pl.BlockSpec((BLK2, OUT), lambda i: (i, 0)),
        out_shape=jax.ShapeDtypeStruct((N_TOK, OUT), f32),
    )(out_pre, acts, r_chunk, qw_p, qb_p)
    return out
